# baseline (speedup 1.0000x reference)
import jax
import jax.numpy as jnp
from jax import lax
from jax.experimental import pallas as pl
from jax.experimental.pallas import tpu as pltpu
from jax.experimental.pallas import tpu_sc as plsc

N = 5000
L = 16
NPAD = 5008
NCHUNK = NPAD // L
MAXK = 128
KSTRIDE = 8
OUTPAD = 512
SCORE_THRESH = 0.05
F32 = jnp.float32
I32 = jnp.int32


def _lane():
    return lax.broadcasted_iota(I32, (L,), 0)


def _bcast(vec, lane_idx):
    return vec.at[jnp.full((L,), lane_idx, I32)].get(mode="promise_in_bounds")


def _row(r):
    return jnp.full((L,), r, I32)


def _nms_body(bh, sh, oh, outh, bv, sv, ov, kbox, outv, sem1, sem2, sem3):
    @pl.when((lax.axis_index("c") == 0) & (lax.axis_index("s") == 0))
    def _():
        cp1 = pltpu.async_copy(bh, bv, sem1)
        cp2 = pltpu.async_copy(sh, sv, sem2)
        cp3 = pltpu.async_copy(oh, ov, sem3)

        zero = jnp.full((L,), 0.0, F32)
        for i in range(OUTPAD // L):
            outv[pl.ds(i * L, L)] = zero

        lane = _lane()
        cp1.wait()
        cp2.wait()
        cp3.wait()

        def chunk_body(st):
            c, K, _go = st
            base = c * L
            idx = ov[pl.ds(base, L)]
            cs = plsc.load_gather(sv, [idx])
            idx4 = idx * 4
            cx1 = plsc.load_gather(bv, [idx4])
            cy1 = plsc.load_gather(bv, [idx4 + 1])
            cx2 = plsc.load_gather(bv, [idx4 + 2])
            cy2 = plsc.load_gather(bv, [idx4 + 3])
            carea = (cx2 - cx1) * (cy2 - cy1)
            valid = (cs > SCORE_THRESH) & (lane + base < N)

            def kept_body(k, supp):
                v = kbox[pl.ds(k * KSTRIDE, L)]
                bx1, by1, bx2, by2, barea = v[0], v[1], v[2], v[3], v[4]
                w = jnp.maximum(
                    jnp.minimum(cx2, bx2) - jnp.maximum(cx1, bx1), 0.0)
                h = jnp.maximum(
                    jnp.minimum(cy2, by2) - jnp.maximum(cy1, by1), 0.0)
                inter = w * h
                union = barea + carea - inter
                return supp | jnp.where(inter > 0.5 * union, 1, 0)

            supp = lax.fori_loop(0, K, kept_body, jnp.full((L,), 0, I32))
            alive = jnp.where(valid & (supp == 0), 1, 0)

            def r_cond(rst):
                _surv, ptr = rst
                return ptr < L

            def r_body(rst):
                surv, ptr = rst
                rem = (surv != 0) & (lane >= ptr)
                l0 = plsc.all_reduce_ffs(rem)[0]
                lsafe = jnp.minimum(l0, L - 1)
                bx1 = _bcast(cx1, lsafe)
                by1 = _bcast(cy1, lsafe)
                bx2 = _bcast(cx2, lsafe)
                by2 = _bcast(cy2, lsafe)
                barea = _bcast(carea, lsafe)
                w = jnp.maximum(
                    jnp.minimum(cx2, bx2) - jnp.maximum(cx1, bx1), 0.0)
                h = jnp.maximum(
                    jnp.minimum(cy2, by2) - jnp.maximum(cy1, by1), 0.0)
                inter = w * h
                union = barea + carea - inter
                kill = (inter > 0.5 * union) & (lane > l0) & (l0 < L)
                return jnp.where(kill, 0, surv), l0 + 1

            surv, _ = lax.while_loop(r_cond, r_body, (alive, jnp.int32(0)))
            survm = surv != 0

            pos = (K + plsc.cumsum(surv) - 1) * KSTRIDE
            plsc.store_scatter(kbox, [pos], cx1, mask=survm)
            plsc.store_scatter(kbox, [pos + 1], cy1, mask=survm)
            plsc.store_scatter(kbox, [pos + 2], cx2, mask=survm)
            plsc.store_scatter(kbox, [pos + 3], cy2, mask=survm)
            plsc.store_scatter(kbox, [pos + 4], carea, mask=survm)
            plsc.store_scatter(kbox, [pos + 5], cs, mask=survm)
            Knew = K + plsc.all_reduce_population_count(survm)[0]

            go = (Knew < 100) & (c + 1 < NCHUNK) & (cs[L - 1] > SCORE_THRESH)
            return c + 1, Knew, go

        def chunk_cond(st):
            _c, _K, go = st
            return go

        _, kfin, _ = lax.while_loop(
            chunk_cond, chunk_body,
            (jnp.int32(0), jnp.int32(0), jnp.bool_(True)))

        for rc in range(7):
            off = rc * L
            rows = lane + off
            m = (rows < 100) & (rows < kfin)
            for col, field in enumerate((0, 1, 2, 3, 5)):
                vals = plsc.load_gather(kbox, [rows * KSTRIDE + field], mask=m)
                plsc.store_scatter(outv, [rows * 5 + col], vals, mask=m)
        pltpu.sync_copy(outv, outh)


@jax.jit
def _nms_sc(boxes, scores, order):
    mesh = plsc.VectorSubcoreMesh(core_axis_name="c", subcore_axis_name="s")
    return pl.kernel(
        _nms_body,
        out_type=jax.ShapeDtypeStruct((OUTPAD,), F32),
        mesh=mesh,
        scratch_types=[pltpu.VMEM((N * 4,), F32),
                       pltpu.VMEM((N,), F32),
                       pltpu.VMEM((NPAD,), I32),
                       pltpu.VMEM((MAXK * KSTRIDE,), F32),
                       pltpu.VMEM((OUTPAD,), F32),
                       pltpu.SemaphoreType.DMA,
                       pltpu.SemaphoreType.DMA,
                       pltpu.SemaphoreType.DMA],
        compiler_params=pltpu.CompilerParams(needs_layout_passes=False),
    )(boxes, scores, order)


def kernel(boxes, scores):
    order = jnp.argsort(-scores).astype(jnp.int32)
    op = jnp.concatenate([order, jnp.zeros((NPAD - N,), jnp.int32)])
    out = _nms_sc(boxes.reshape(-1), scores, op)
    return out[:500].reshape(100, 5)


# speedup vs baseline: 1.1091x; 1.1091x over previous
import jax
import jax.numpy as jnp
from jax import lax
from jax.experimental import pallas as pl
from jax.experimental.pallas import tpu as pltpu
from jax.experimental.pallas import tpu_sc as plsc

N = 5000
L = 16
NPAD = 5008
NCHUNK = NPAD // L
MAXK = 128
KSTRIDE = 8
OUTPAD = 512
SCORE_THRESH = 0.05
F32 = jnp.float32
I32 = jnp.int32


def _lane():
    return lax.broadcasted_iota(I32, (L,), 0)


def _bcast(vec, lane_idx):
    return vec.at[jnp.full((L,), lane_idx, I32)].get(mode="promise_in_bounds")


def _row(r):
    return jnp.full((L,), r, I32)


def _nms_body(ph, oh, outh, pv, ov, kbox, outv, sem1, sem2):
    @pl.when((lax.axis_index("c") == 0) & (lax.axis_index("s") == 0))
    def _():
        cp1 = pltpu.async_copy(ph, pv, sem1)
        cp2 = pltpu.async_copy(oh, ov, sem2)

        zero = jnp.full((L,), 0.0, F32)
        for i in range(OUTPAD // L):
            outv[pl.ds(i * L, L)] = zero

        lane = _lane()
        cp1.wait()
        cp2.wait()

        def chunk_body(st):
            c, K, _go = st
            base = c * L
            idx = ov[pl.ds(base, L)]
            cs = plsc.load_gather(pv, [_row(4), idx])
            cx1 = plsc.load_gather(pv, [_row(0), idx])
            cy1 = plsc.load_gather(pv, [_row(1), idx])
            cx2 = plsc.load_gather(pv, [_row(2), idx])
            cy2 = plsc.load_gather(pv, [_row(3), idx])
            carea = (cx2 - cx1) * (cy2 - cy1)
            valid = cs > SCORE_THRESH

            def _iou_kill(bx1, by1, bx2, by2, barea):
                w = jnp.maximum(
                    jnp.minimum(cx2, bx2) - jnp.maximum(cx1, bx1), 0.0)
                h = jnp.maximum(
                    jnp.minimum(cy2, by2) - jnp.maximum(cy1, by1), 0.0)
                inter = w * h
                union = barea + carea - inter
                return inter > 0.5 * union

            def kept_body(k2, supp):
                v = kbox[pl.ds(k2 * (2 * KSTRIDE), L)]
                sa = _iou_kill(v[0], v[1], v[2], v[3], v[4])
                sb = _iou_kill(v[8], v[9], v[10], v[11], v[12])
                sb = sb & (2 * k2 + 1 < K)
                return supp | jnp.where(sa | sb, 1, 0)

            supp = lax.fori_loop(0, (K + 1) // 2, kept_body,
                                 jnp.full((L,), 0, I32))
            alive = jnp.where(valid & (supp == 0), 1, 0)

            def r_cond(rst):
                _surv, ptr = rst
                return ptr < L

            def r_body(rst):
                surv, ptr = rst
                rem = (surv != 0) & (lane >= ptr)
                l0 = plsc.all_reduce_ffs(rem)[0]
                lsafe = jnp.minimum(l0, L - 1)
                kill = _iou_kill(_bcast(cx1, lsafe), _bcast(cy1, lsafe),
                                 _bcast(cx2, lsafe), _bcast(cy2, lsafe),
                                 _bcast(carea, lsafe))
                kill = kill & (lane > l0) & (l0 < L)
                return jnp.where(kill, 0, surv), l0 + 1

            surv, _ = lax.while_loop(r_cond, r_body, (alive, jnp.int32(0)))
            survm = surv != 0

            pos = (K + plsc.cumsum(surv) - 1) * KSTRIDE
            plsc.store_scatter(kbox, [pos], cx1, mask=survm)
            plsc.store_scatter(kbox, [pos + 1], cy1, mask=survm)
            plsc.store_scatter(kbox, [pos + 2], cx2, mask=survm)
            plsc.store_scatter(kbox, [pos + 3], cy2, mask=survm)
            plsc.store_scatter(kbox, [pos + 4], carea, mask=survm)
            plsc.store_scatter(kbox, [pos + 5], cs, mask=survm)
            Knew = K + plsc.all_reduce_population_count(survm)[0]

            go = (Knew < 100) & (c + 1 < NCHUNK) & (cs[L - 1] > SCORE_THRESH)
            return c + 1, Knew, go

        def chunk_cond(st):
            _c, _K, go = st
            return go

        _, kfin, _ = lax.while_loop(
            chunk_cond, chunk_body,
            (jnp.int32(0), jnp.int32(0), jnp.bool_(True)))

        for rc in range(7):
            off = rc * L
            rows = lane + off
            m = (rows < 100) & (rows < kfin)
            for col, field in enumerate((0, 1, 2, 3, 5)):
                vals = plsc.load_gather(kbox, [rows * KSTRIDE + field], mask=m)
                plsc.store_scatter(outv, [rows * 5 + col], vals, mask=m)
        pltpu.sync_copy(outv, outh)


@jax.jit
def _nms_sc(packed, order):
    mesh = plsc.VectorSubcoreMesh(core_axis_name="c", subcore_axis_name="s",
                                  num_cores=1)
    return pl.kernel(
        _nms_body,
        out_type=jax.ShapeDtypeStruct((OUTPAD,), F32),
        mesh=mesh,
        scratch_types=[pltpu.VMEM((5, NPAD), F32),
                       pltpu.VMEM((NPAD,), I32),
                       pltpu.VMEM((MAXK * KSTRIDE,), F32),
                       pltpu.VMEM((OUTPAD,), F32),
                       pltpu.SemaphoreType.DMA,
                       pltpu.SemaphoreType.DMA],
        compiler_params=pltpu.CompilerParams(needs_layout_passes=False),
    )(packed, order)


def kernel(boxes, scores):
    order = jnp.argsort(-scores).astype(jnp.int32)
    op = jnp.concatenate([order, jnp.arange(N, NPAD, dtype=jnp.int32)])
    packed = jnp.concatenate(
        [boxes.T, scores[None, :]], axis=0)
    packed = jnp.pad(packed, ((0, 0), (0, NPAD - N)))
    out = _nms_sc(packed, op)
    return out[:500].reshape(100, 5)


# speedup vs baseline: 1.1916x; 1.0744x over previous
import jax
import jax.numpy as jnp
from jax import lax
from jax.experimental import pallas as pl
from jax.experimental.pallas import tpu as pltpu
from jax.experimental.pallas import tpu_sc as plsc

N = 5000
L = 16
NPAD = 5008
NCHUNK = NPAD // L
MAXK = 128
KSTRIDE = 8
OUTPAD = 512
SCORE_THRESH = 0.05
F32 = jnp.float32
I32 = jnp.int32


def _lane():
    return lax.broadcasted_iota(I32, (L,), 0)


def _bcast(vec, lane_idx):
    return vec.at[jnp.full((L,), lane_idx, I32)].get(mode="promise_in_bounds")


def _row(r):
    return jnp.full((L,), r, I32)


def _nms_body(ph, oh, outh, pv, ov, kbox, outv, sem1, sem2):
    @pl.when((lax.axis_index("c") == 0) & (lax.axis_index("s") == 0))
    def _():
        cp1 = pltpu.async_copy(ph, pv, sem1)
        cp2 = pltpu.async_copy(oh, ov, sem2)

        zero = jnp.full((L,), 0.0, F32)
        for i in range(OUTPAD // L):
            outv[pl.ds(i * L, L)] = zero

        lane = _lane()
        cp1.wait()
        cp2.wait()

        def chunk_body(st):
            c, K, _go = st
            base = c * L
            idx = ov[pl.ds(base, L)]
            cs = plsc.load_gather(pv, [_row(4), idx])
            cx1 = plsc.load_gather(pv, [_row(0), idx])
            cy1 = plsc.load_gather(pv, [_row(1), idx])
            cx2 = plsc.load_gather(pv, [_row(2), idx])
            cy2 = plsc.load_gather(pv, [_row(3), idx])
            carea = (cx2 - cx1) * (cy2 - cy1)
            valid = cs > SCORE_THRESH

            def _iou_kill(bx1, by1, bx2, by2, barea):
                w = jnp.maximum(
                    jnp.minimum(cx2, bx2) - jnp.maximum(cx1, bx1), 0.0)
                h = jnp.maximum(
                    jnp.minimum(cy2, by2) - jnp.maximum(cy1, by1), 0.0)
                inter = w * h
                union = barea + carea - inter
                return inter > 0.5 * union

            def kept_body(k2, supp):
                v = kbox[pl.ds(k2 * (2 * KSTRIDE), L)]
                sa = _iou_kill(v[0], v[1], v[2], v[3], v[4])
                sb = _iou_kill(v[8], v[9], v[10], v[11], v[12])
                sb = sb & (2 * k2 + 1 < K)
                return supp | jnp.where(sa | sb, 1, 0)

            supp = lax.fori_loop(0, (K + 1) // 2, kept_body,
                                 jnp.full((L,), 0, I32))
            alive = jnp.where(valid & (supp == 0), 1, 0)

            kills = []
            for l in range(L - 1):
                k = _iou_kill(_bcast(cx1, l), _bcast(cy1, l),
                              _bcast(cx2, l), _bcast(cy2, l),
                              _bcast(carea, l))
                kills.append(k & (lane > l))
            surv = alive
            for l in range(L - 1):
                on = _bcast(surv, l) != 0
                surv = jnp.where(kills[l] & on, 0, surv)
            survm = surv != 0

            pos = (K + plsc.cumsum(surv) - 1) * KSTRIDE
            plsc.store_scatter(kbox, [pos], cx1, mask=survm)
            plsc.store_scatter(kbox, [pos + 1], cy1, mask=survm)
            plsc.store_scatter(kbox, [pos + 2], cx2, mask=survm)
            plsc.store_scatter(kbox, [pos + 3], cy2, mask=survm)
            plsc.store_scatter(kbox, [pos + 4], carea, mask=survm)
            plsc.store_scatter(kbox, [pos + 5], cs, mask=survm)
            Knew = K + plsc.all_reduce_population_count(survm)[0]

            go = (Knew < 100) & (c + 1 < NCHUNK) & (cs[L - 1] > SCORE_THRESH)
            return c + 1, Knew, go

        def chunk_cond(st):
            _c, _K, go = st
            return go

        _, kfin, _ = lax.while_loop(
            chunk_cond, chunk_body,
            (jnp.int32(0), jnp.int32(0), jnp.bool_(True)))

        for rc in range(7):
            off = rc * L
            rows = lane + off
            m = (rows < 100) & (rows < kfin)
            for col, field in enumerate((0, 1, 2, 3, 5)):
                vals = plsc.load_gather(kbox, [rows * KSTRIDE + field], mask=m)
                plsc.store_scatter(outv, [rows * 5 + col], vals, mask=m)
        pltpu.sync_copy(outv, outh)


@jax.jit
def _nms_sc(packed, order):
    mesh = plsc.VectorSubcoreMesh(core_axis_name="c", subcore_axis_name="s",
                                  num_cores=1)
    return pl.kernel(
        _nms_body,
        out_type=jax.ShapeDtypeStruct((OUTPAD,), F32),
        mesh=mesh,
        scratch_types=[pltpu.VMEM((5, NPAD), F32),
                       pltpu.VMEM((NPAD,), I32),
                       pltpu.VMEM((MAXK * KSTRIDE,), F32),
                       pltpu.VMEM((OUTPAD,), F32),
                       pltpu.SemaphoreType.DMA,
                       pltpu.SemaphoreType.DMA],
        compiler_params=pltpu.CompilerParams(needs_layout_passes=False),
    )(packed, order)


def kernel(boxes, scores):
    order = jnp.argsort(-scores).astype(jnp.int32)
    op = jnp.concatenate([order, jnp.arange(N, NPAD, dtype=jnp.int32)])
    packed = jnp.concatenate(
        [boxes.T, scores[None, :]], axis=0)
    packed = jnp.pad(packed, ((0, 0), (0, NPAD - N)))
    out = _nms_sc(packed, op)
    return out[:500].reshape(100, 5)
